# Initial kernel scaffold; baseline (speedup 1.0000x reference)
import jax
import jax.numpy as jnp
from jax.experimental import pallas as pl


def kernel(inputs, y):
    raise NotImplementedError("write your pallas kernel here")


# speedup vs baseline: 9.1453x; 9.1453x over previous
import jax
import jax.numpy as jnp
from jax import lax
from jax.experimental import pallas as pl

_NUM_UNITS = 1024
_SCALE = float(_NUM_UNITS) ** 0.5


def _posenc_block_kernel(out_ref):
    bt = out_ref.shape[1]
    d = out_ref.shape[2]
    t0 = pl.program_id(0) * bt
    pos = (lax.broadcasted_iota(jnp.int32, (bt, d), 0) + t0).astype(jnp.float32)
    col = lax.broadcasted_iota(jnp.int32, (bt, d), 1)
    half = (col // 2).astype(jnp.float32)
    denom = jnp.exp(half * jnp.float32(2.0 / _NUM_UNITS * 9.210340371976184))
    angle = pos / denom
    even = (col % 2) == 0
    enc = jnp.where(even, jnp.sin(angle), jnp.cos(angle))
    enc = jnp.where(pos == 0.0, 0.0, enc) * jnp.float32(_SCALE)
    out_ref[...] = jnp.broadcast_to(enc[None], out_ref.shape)


def kernel(inputs, y):
    n, t, d = inputs.shape
    bt = 256
    out = pl.pallas_call(
        _posenc_block_kernel,
        grid=(t // bt,),
        out_specs=pl.BlockSpec((n, bt, d), lambda i: (0, i, 0)),
        out_shape=jax.ShapeDtypeStruct((n, t, d), jnp.float32),
    )()
    return out


# speedup vs baseline: 28.0025x; 3.0620x over previous
import jax
import jax.numpy as jnp
from jax import lax
from jax.experimental import pallas as pl

_NUM_UNITS = 1024
_SCALE = float(_NUM_UNITS) ** 0.5
_LN1E4 = 9.210340371976184
_SUB = 8


def _posenc_block_kernel(out_ref):
    n, bt, d = out_ref.shape
    t0 = pl.program_id(0) * bt

    col = lax.broadcasted_iota(jnp.int32, (_SUB, d), 1)
    half = (col // 2).astype(jnp.float32)
    w = jnp.exp(half * jnp.float32(-2.0 * _LN1E4 / _NUM_UNITS))
    even = (col % 2) == 0

    pos = (lax.broadcasted_iota(jnp.int32, (_SUB, d), 0) + t0).astype(jnp.float32)
    a = pos * w
    s = jnp.sin(a)
    c = jnp.cos(a)
    A = jnp.where(even, s, c) * jnp.float32(_SCALE)
    B = jnp.where(even, c, s) * jnp.float32(_SCALE)

    a8 = jnp.float32(_SUB) * w
    c8 = jnp.cos(a8)
    s8 = jnp.sin(a8)
    s8s = jnp.where(even, s8, -s8)

    A_store = jnp.where(pos == 0.0, 0.0, A)
    out_ref[:, 0:_SUB, :] = jnp.broadcast_to(A_store[None], (n, _SUB, d))

    for j in range(1, bt // _SUB):
        A, B = A * c8 + B * s8s, B * c8 - A * s8s
        out_ref[:, j * _SUB:(j + 1) * _SUB, :] = jnp.broadcast_to(
            A[None], (n, _SUB, d)
        )


def kernel(inputs, y):
    n, t, d = inputs.shape
    bt = 256
    out = pl.pallas_call(
        _posenc_block_kernel,
        grid=(t // bt,),
        out_specs=pl.BlockSpec((n, bt, d), lambda i: (0, i, 0)),
        out_shape=jax.ShapeDtypeStruct((n, t, d), jnp.float32),
    )()
    return out
